# Initial kernel scaffold; baseline (speedup 1.0000x reference)
import jax
import jax.numpy as jnp
from jax.experimental import pallas as pl


def kernel(x, adj, W, a, W_mu, a_mu, W_lv, a_lv):
    raise NotImplementedError("write your pallas kernel here")


# speedup vs baseline: 30.9849x; 30.9849x over previous
import functools

import jax
import jax.numpy as jnp
import numpy as np
from jax.experimental import pallas as pl

N = 10000
NFEAT = 128
NHID = 8
NOUT = 16
NHEADS = 8
ALPHA = 0.2

BI = 400
NI = N // BI
BIA = 80
NIA = N // BIA


def _k1_body(x_ref, wc_ref, a1_ref, a2_ref,
             u_ref, ua_ref, v_ref, va_ref, haug_ref):
    h = jnp.dot(x_ref[...], wc_ref[...], preferred_element_type=jnp.float32)
    s1 = jnp.dot(h, a1_ref[...], preferred_element_type=jnp.float32)
    s2 = jnp.dot(h, a2_ref[...], preferred_element_type=jnp.float32)
    u_ref[...] = jnp.exp(-s1)
    ua_ref[...] = jnp.exp(-ALPHA * s1)
    v_ref[...] = jnp.exp(-s2)
    va_ref[...] = jnp.exp(-ALPHA * s2)
    ones = jnp.ones((h.shape[0], 1), jnp.float32)
    parts = []
    for k in range(NHEADS):
        parts.append(h[:, k * NHID:(k + 1) * NHID])
        parts.append(ones)
    haug_ref[...] = jnp.concatenate(parts, axis=1)


def _att_body(adj_ref, u_ref, ua_ref, vt_ref, vat_ref, haug_ref, out_ref,
              *, nheads, width):
    adjf = adj_ref[...].astype(jnp.float32)
    w = width + 1
    outs = []
    for k in range(nheads):
        p1 = u_ref[:, k:k + 1] * vt_ref[k:k + 1, :]
        p2 = ua_ref[:, k:k + 1] * vat_ref[k:k + 1, :]
        e = jnp.minimum(p1, p2) * adjf
        outs.append(jnp.dot(e, haug_ref[:, k * w:(k + 1) * w],
                            preferred_element_type=jnp.float32))
    out_ref[...] = jnp.concatenate(outs, axis=1)


def _att_pass(adj, u, ua, vt, vat, haug, nheads, width):
    w = width + 1
    body = functools.partial(_att_body, nheads=nheads, width=width)
    return pl.pallas_call(
        body,
        grid=(NIA,),
        in_specs=[
            pl.BlockSpec((BIA, N), lambda i: (i, 0)),
            pl.BlockSpec((BIA, nheads), lambda i: (i, 0)),
            pl.BlockSpec((BIA, nheads), lambda i: (i, 0)),
            pl.BlockSpec((nheads, N), lambda i: (0, 0)),
            pl.BlockSpec((nheads, N), lambda i: (0, 0)),
            pl.BlockSpec((N, nheads * w), lambda i: (0, 0)),
        ],
        out_specs=pl.BlockSpec((BIA, nheads * w), lambda i: (i, 0)),
        out_shape=jax.ShapeDtypeStruct((N, nheads * w), jnp.float32),
    )(adj, u, ua, vt, vat, haug)


def _k3_body(acc_ref, wml_ref, b1_ref, b2_ref,
             u_ref, ua_ref, v_ref, va_ref, gaug_ref):
    acc = acc_ref[...]
    hs = []
    for k in range(NHEADS):
        num = acc[:, k * 9:k * 9 + NHID]
        den = acc[:, k * 9 + NHID:k * 9 + NHID + 1]
        hp = num / den
        hs.append(jnp.where(hp > 0, hp, jnp.exp(hp) - 1.0))
    h1 = jnp.concatenate(hs, axis=1)
    g = jnp.dot(h1, wml_ref[...], preferred_element_type=jnp.float32)
    s1 = jnp.dot(g, b1_ref[...], preferred_element_type=jnp.float32)
    s2 = jnp.dot(g, b2_ref[...], preferred_element_type=jnp.float32)
    u_ref[...] = jnp.exp(-s1)
    ua_ref[...] = jnp.exp(-ALPHA * s1)
    v_ref[...] = jnp.exp(-s2)
    va_ref[...] = jnp.exp(-ALPHA * s2)
    ones = jnp.ones((g.shape[0], 1), jnp.float32)
    gaug_ref[...] = jnp.concatenate(
        [g[:, :NOUT], ones, g[:, NOUT:], ones], axis=1)


def _k5_body(acc_ref, mu_ref, lv_ref):
    acc = acc_ref[...]
    mu_ref[...] = acc[:, 0:NOUT] / acc[:, NOUT:NOUT + 1]
    lv_ref[...] = acc[:, NOUT + 1:2 * NOUT + 1] / acc[:, 2 * NOUT + 1:]


def kernel(x, adj, W, a, W_mu, a_mu, W_lv, a_lv):
    f32 = jnp.float32
    wc = jnp.transpose(W, (1, 0, 2)).reshape(NFEAT, NHEADS * NHID)
    eye = jnp.eye(NHEADS, dtype=f32)
    a1 = (a[:, 0, :NHID][:, :, None] * eye[:, None, :]).reshape(
        NHEADS * NHID, NHEADS)
    a2 = (a[:, 0, NHID:][:, :, None] * eye[:, None, :]).reshape(
        NHEADS * NHID, NHEADS)

    k1 = pl.pallas_call(
        _k1_body,
        grid=(NI,),
        in_specs=[
            pl.BlockSpec((BI, NFEAT), lambda i: (i, 0)),
            pl.BlockSpec((NFEAT, NHEADS * NHID), lambda i: (0, 0)),
            pl.BlockSpec((NHEADS * NHID, NHEADS), lambda i: (0, 0)),
            pl.BlockSpec((NHEADS * NHID, NHEADS), lambda i: (0, 0)),
        ],
        out_specs=[
            pl.BlockSpec((BI, NHEADS), lambda i: (i, 0)),
            pl.BlockSpec((BI, NHEADS), lambda i: (i, 0)),
            pl.BlockSpec((BI, NHEADS), lambda i: (i, 0)),
            pl.BlockSpec((BI, NHEADS), lambda i: (i, 0)),
            pl.BlockSpec((BI, NHEADS * 9), lambda i: (i, 0)),
        ],
        out_shape=[
            jax.ShapeDtypeStruct((N, NHEADS), f32),
            jax.ShapeDtypeStruct((N, NHEADS), f32),
            jax.ShapeDtypeStruct((N, NHEADS), f32),
            jax.ShapeDtypeStruct((N, NHEADS), f32),
            jax.ShapeDtypeStruct((N, NHEADS * 9), f32),
        ],
    )(x, wc, a1, a2)
    u, ua, v, va, haug = k1

    adj8 = adj.astype(jnp.int8)
    acc1 = _att_pass(adj8, u, ua, v.T, va.T, haug, NHEADS, NHID)

    wml = jnp.concatenate([W_mu, W_lv], axis=1)
    z2 = jnp.zeros((NOUT, 1), f32)
    b1 = jnp.concatenate([
        jnp.concatenate([a_mu[0, :NOUT, None], z2], axis=1),
        jnp.concatenate([z2, a_lv[0, :NOUT, None]], axis=1)], axis=0)
    b2 = jnp.concatenate([
        jnp.concatenate([a_mu[0, NOUT:, None], z2], axis=1),
        jnp.concatenate([z2, a_lv[0, NOUT:, None]], axis=1)], axis=0)

    k3 = pl.pallas_call(
        _k3_body,
        grid=(NI,),
        in_specs=[
            pl.BlockSpec((BI, NHEADS * 9), lambda i: (i, 0)),
            pl.BlockSpec((NHEADS * NHID, 2 * NOUT), lambda i: (0, 0)),
            pl.BlockSpec((2 * NOUT, 2), lambda i: (0, 0)),
            pl.BlockSpec((2 * NOUT, 2), lambda i: (0, 0)),
        ],
        out_specs=[
            pl.BlockSpec((BI, 2), lambda i: (i, 0)),
            pl.BlockSpec((BI, 2), lambda i: (i, 0)),
            pl.BlockSpec((BI, 2), lambda i: (i, 0)),
            pl.BlockSpec((BI, 2), lambda i: (i, 0)),
            pl.BlockSpec((BI, 2 * (NOUT + 1)), lambda i: (i, 0)),
        ],
        out_shape=[
            jax.ShapeDtypeStruct((N, 2), f32),
            jax.ShapeDtypeStruct((N, 2), f32),
            jax.ShapeDtypeStruct((N, 2), f32),
            jax.ShapeDtypeStruct((N, 2), f32),
            jax.ShapeDtypeStruct((N, 2 * (NOUT + 1)), f32),
        ],
    )(acc1, wml, b1, b2)
    u2, ua2, v2, va2, gaug = k3

    acc2 = _att_pass(adj8, u2, ua2, v2.T, va2.T, gaug, 2, NOUT)

    mu, lv = pl.pallas_call(
        _k5_body,
        grid=(NI,),
        in_specs=[pl.BlockSpec((BI, 2 * (NOUT + 1)), lambda i: (i, 0))],
        out_specs=[
            pl.BlockSpec((BI, NOUT), lambda i: (i, 0)),
            pl.BlockSpec((BI, NOUT), lambda i: (i, 0)),
        ],
        out_shape=[
            jax.ShapeDtypeStruct((N, NOUT), f32),
            jax.ShapeDtypeStruct((N, NOUT), f32),
        ],
    )(acc2)

    return (mu, mu, lv)
